# Initial kernel scaffold; baseline (speedup 1.0000x reference)
import jax
import jax.numpy as jnp
from jax.experimental import pallas as pl


def kernel(x, edge_vec, t, x_atm, edge_index, batch, emb_x, emb_z, gfp_W, t_W1, t_b1, t_W2, t_b2, sc_W0, msg_W0, sh_W0, z_W0, rad_W1_0, rad_b1_0, rad_W2_0, rad_b2_0, sc_W1, msg_W1, sh_W1, z_W1, rad_W1_1, rad_b1_1, rad_W2_1, rad_b2_1, sc_W2, msg_W2, sh_W2, z_W2, rad_W1_2, rad_b1_2, rad_W2_2, rad_b2_2, W_out):
    raise NotImplementedError("write your pallas kernel here")


# speedup vs baseline: 1.3411x; 1.3411x over previous
import numpy as np
import jax
import jax.numpy as jnp
from jax import lax
from jax.experimental import pallas as pl
from jax.experimental.pallas import tpu as pltpu
from jax.experimental.pallas import tpu_sc as plsc

N = 10000
E = 160000
B = 128
IN0 = 19
HID = 320
GIN = 384
NUM_BASIS = 32
MAX_RADIUS = 5.0
NUM_NEIGHBORS = 12.0

NC = 2
NS = 16
NP = 3
FC = GIN // (NC * NP)
ET = E // NS
CHUNK = 125
NCHUNK = ET // CHUNK
NPAD = 10240
ROWS_PER_TILE = NPAD // NS
ZROWS = 128

_EB = 2000
_NB = 2000


def _sigmoid(x):
    return 1.0 / (1.0 + jnp.exp(-x))


def _silu(x):
    return x * _sigmoid(x)


def _sus(u):
    return jnp.where(u > 0.0, jnp.exp(-1.0 / jnp.where(u > 0.0, u, 1.0)), 0.0)


def _te_body(t_ref, gfp_ref, w1_ref, b1_ref, w2_ref, b2_ref, out_ref):
    proj = t_ref[...] * gfp_ref[...] * (2.0 * np.pi)
    te = jnp.concatenate([jnp.sin(proj), jnp.cos(proj)], axis=1)
    te = _silu(te @ w1_ref[...] + b1_ref[...]) @ w2_ref[...] + b2_ref[...]
    out_ref[...] = te


def _time_embed(t, gfp_W, t_W1, t_b1, t_W2, t_b2):
    return pl.pallas_call(
        _te_body,
        out_shape=jax.ShapeDtypeStruct((B, 64), jnp.float32),
    )(t.reshape(B, 1), gfp_W.reshape(1, 64), t_W1, t_b1.reshape(1, 128),
      t_W2, t_b2.reshape(1, 64))


def _edge_body(ev_ref, rw1_ref, rb1_ref, rw2_ref, rb2_ref, shw_ref, out_ref):
    ev = ev_ref[...]
    r = jnp.sqrt(jnp.sum(ev * ev, axis=1, keepdims=True))
    step = MAX_RADIUS / (NUM_BASIS + 1)
    centers = (lax.broadcasted_iota(jnp.int32, (1, NUM_BASIS), 1)
               .astype(jnp.float32) + 1.0) * step
    diff = (r - centers) / step
    hb = (1.14136 * float(np.exp(2.0)) * (NUM_BASIS ** 0.5)) * \
        _sus(diff + 1.0) * _sus(1.0 - diff)
    wr = _silu(hb @ rw1_ref[...] + rb1_ref[...]) @ rw2_ref[...] + rb2_ref[...]
    u = ev / jnp.maximum(r, 1e-9)
    ux, uy, uz = u[:, 0:1], u[:, 1:2], u[:, 2:3]
    s3 = 3.0 ** 0.5
    s5 = 5.0 ** 0.5
    s15 = 15.0 ** 0.5
    cols = (jnp.ones_like(ux), s3 * ux, s3 * uy, s3 * uz,
            s15 * ux * uz, s15 * ux * uy,
            s5 * (uy * uy - 0.5 * (ux * ux + uz * uz)),
            s15 * uy * uz, 0.5 * s15 * (uz * uz - ux * ux))
    shw = shw_ref[...]
    acc = cols[0] * shw[0:1, :]
    for q in range(1, 9):
        acc = acc + cols[q] * shw[q:q + 1, :]
    wp = wr * acc
    for q in range(NC * NP):
        out_ref[q] = wp[:, q * FC:(q + 1) * FC].astype(jnp.bfloat16)


def _edge_wprod(edge_vec, rW1, rb1, rW2, rb2, sh_W):
    grid = E // _EB
    full = lambda i: (0, 0)
    return pl.pallas_call(
        _edge_body,
        grid=(grid,),
        in_specs=[
            pl.BlockSpec((_EB, 3), lambda i: (i, 0)),
            pl.BlockSpec((NUM_BASIS, 64), full),
            pl.BlockSpec((1, 64), full),
            pl.BlockSpec((64, GIN), full),
            pl.BlockSpec((1, GIN), full),
            pl.BlockSpec((9, GIN), full),
        ],
        out_specs=pl.BlockSpec((NC * NP, _EB, FC), lambda i: (0, i, 0)),
        out_shape=jax.ShapeDtypeStruct((NC * NP, E, FC), jnp.bfloat16),
    )(edge_vec, rW1, rb1.reshape(1, 64), rW2, rb2.reshape(1, GIN), sh_W)


def _hm0_body(h_ref, mw_ref, out_ref):
    hm = h_ref[...] @ mw_ref[...]
    for q in range(NC * NP):
        out_ref[q] = hm[:, q * FC:(q + 1) * FC].astype(jnp.bfloat16)


def _hm0(h0, msg_W0):
    grid = N // _NB
    return pl.pallas_call(
        _hm0_body,
        grid=(grid,),
        in_specs=[
            pl.BlockSpec((_NB, IN0), lambda i: (i, 0)),
            pl.BlockSpec((IN0, GIN), lambda i: (0, 0)),
        ],
        out_specs=pl.BlockSpec((NC * NP, _NB, FC), lambda i: (0, i, 0)),
        out_shape=jax.ShapeDtypeStruct((NC * NP, N, FC), jnp.bfloat16),
    )(h0, msg_W0)


def _node_update(h_ref, b_ref, te_ref, ez_ref, agg_refs,
                 scw_ref, zw_ref, s1_ref, s2_ref):
    lanes = lax.broadcasted_iota(jnp.int32, (1, B), 1)
    onehot = (b_ref[...] == lanes).astype(jnp.float32)
    z = onehot @ te_ref[...] + ez_ref[...]
    agg = jnp.concatenate([a[...] for a in agg_refs], axis=1) * \
        (1.0 / (NUM_NEIGHBORS ** 0.5))
    pre = h_ref[...] @ scw_ref[...] + z @ zw_ref[...] + agg
    scal = _silu(pre[:, :64])
    gates = _sigmoid(pre[:, 64:128])
    gated = pre[:, 128:]
    g1 = gates[:, :32] @ s1_ref[...]
    g2 = gates[:, 32:64] @ s2_ref[...]
    hn = jnp.concatenate(
        [scal, gated[:, :96] * g1, gated[:, 96:] * g2], axis=1)
    return z, hn


def _node_mid_body(h_ref, b_ref, te_ref, ez_ref, a0, a1, a2, a3, a4, a5,
                   scw_ref, zw_ref, s1_ref, s2_ref, mw_ref,
                   hn_ref, hm_ref):
    _, hn = _node_update(h_ref, b_ref, te_ref, ez_ref,
                         (a0, a1, a2, a3, a4, a5),
                         scw_ref, zw_ref, s1_ref, s2_ref)
    hn_ref[...] = hn
    hm = hn @ mw_ref[...]
    for q in range(NC * NP):
        hm_ref[q] = hm[:, q * FC:(q + 1) * FC].astype(jnp.bfloat16)


def _node_last_body(h_ref, b_ref, te_ref, ez_ref, a0, a1, a2, a3, a4, a5,
                    scw_ref, zw_ref, s1_ref, s2_ref, wp_ref, out_ref):
    z, hn = _node_update(h_ref, b_ref, te_ref, ez_ref,
                         (a0, a1, a2, a3, a4, a5),
                         scw_ref, zw_ref, s1_ref, s2_ref)
    outs = []
    for k in range(3):
        tk = hn @ wp_ref[k]
        outs.append(jnp.sum(tk * z, axis=1, keepdims=True))
    out_ref[...] = jnp.concatenate(outs, axis=1)


def _node_specs(d):
    full = lambda i: (0, 0)
    return [
        pl.BlockSpec((_NB, d), lambda i: (i, 0)),
        pl.BlockSpec((_NB, 1), lambda i: (i, 0)),
        pl.BlockSpec((B, 64), full),
        pl.BlockSpec((1, 64), full),
    ] + [pl.BlockSpec((_NB, FC), lambda i: (i, 0))] * (NC * NP) + [
        pl.BlockSpec((d, GIN), full),
        pl.BlockSpec((64, GIN), full),
        pl.BlockSpec((32, 96), full),
        pl.BlockSpec((32, 160), full),
    ]


def _node_mid(h, batch2, te, emb_z, aggs, sc_W, z_W, S1, S2, msg_W_next):
    d = h.shape[1]
    grid = N // _NB
    return pl.pallas_call(
        _node_mid_body,
        grid=(grid,),
        in_specs=_node_specs(d) + [pl.BlockSpec((HID, GIN), lambda i: (0, 0))],
        out_specs=[
            pl.BlockSpec((_NB, HID), lambda i: (i, 0)),
            pl.BlockSpec((NC * NP, _NB, FC), lambda i: (0, i, 0)),
        ],
        out_shape=[
            jax.ShapeDtypeStruct((N, HID), jnp.float32),
            jax.ShapeDtypeStruct((NC * NP, N, FC), jnp.bfloat16),
        ],
    )(h, batch2, te, emb_z, *aggs, sc_W, z_W, S1, S2, msg_W_next)


def _node_last(h, batch2, te, emb_z, aggs, sc_W, z_W, S1, S2, W_perm):
    d = h.shape[1]
    grid = N // _NB
    return pl.pallas_call(
        _node_last_body,
        grid=(grid,),
        in_specs=_node_specs(d) + [pl.BlockSpec((3, HID, 64), lambda i: (0, 0, 0))],
        out_specs=pl.BlockSpec((_NB, 3), lambda i: (i, 0)),
        out_shape=jax.ShapeDtypeStruct((N, 3), jnp.float32),
    )(h, batch2, te, emb_z, *aggs, sc_W, z_W, S1, S2, W_perm)


def _sc_body(src_hbm, dst_hbm, hm_hbm, wp_hbm, out,
             idxs, idxd, rows, wbuf, prod, zbuf, agg_sh, semg, semw, sems):
    c = lax.axis_index("c")
    s = lax.axis_index("s")
    rs = pl.ds(s * ROWS_PER_TILE, ROWS_PER_TILE)

    pltpu.sync_copy(dst_hbm.at[s], idxd)

    def _zrow(j, carry):
        for k in range(FC // 16):
            zbuf[j, pl.ds(k * 16, 16)] = jnp.zeros((16,), jnp.float32)
        return carry
    lax.fori_loop(0, ZROWS, _zrow, 0)

    for p in range(NP):
        pltpu.sync_copy(src_hbm.at[c, p, s], idxs)
        for i in range(ROWS_PER_TILE // ZROWS):
            pltpu.sync_copy(
                zbuf, agg_sh.at[pl.ds(s * ROWS_PER_TILE + i * ZROWS, ZROWS)])
        plsc.subcore_barrier()

        ebase = (c * NP + p) * E + s * ET

        def _issue(i, b):
            pltpu.async_copy(hm_hbm.at[idxs.at[i]], rows.at[b], semg[b])
            pltpu.async_copy(wp_hbm.at[pl.ds(ebase + i * CHUNK, CHUNK)],
                             wbuf.at[b], semw[b])

        def _wait(b):
            pltpu.make_async_copy(hm_hbm.at[pl.ds(0, CHUNK)], rows.at[b],
                                  semg[b]).wait()
            pltpu.make_async_copy(wp_hbm.at[pl.ds(0, CHUNK)], wbuf.at[b],
                                  semw[b]).wait()

        def _process(i, b):
            _wait(b)

            @plsc.parallel_loop(0, CHUNK, unroll=8)
            def _mul(j):
                for k in range(FC // 32):
                    sl = pl.ds(k * 32, 32)
                    pr = rows[b, j, sl] * wbuf[b, j, sl]
                    lo, hi = plsc.unpack(
                        pr, format=plsc.PackFormat.INTERLEAVED)
                    prod[b, j, pl.ds(k * 32, 16)] = lo
                    prod[b, j, pl.ds(k * 32 + 16, 16)] = hi
            pltpu.async_copy(prod.at[b], agg_sh.at[idxd.at[i]], sems[b],
                             add=True)

        def _wait_scatter(b):
            pltpu.make_async_copy(prod.at[b], agg_sh.at[pl.ds(0, CHUNK)],
                                  sems[b]).wait()

        _issue(0, 0)

        def _pair(g, carry):
            _issue(2 * g + 1, 1)
            _process(2 * g, 0)

            @pl.when(g < NCHUNK // 2 - 1)
            def _():
                _wait_scatter(0)
                _issue(2 * g + 2, 0)
            _process(2 * g + 1, 1)

            @pl.when(g < NCHUNK // 2 - 1)
            def _():
                _wait_scatter(1)
            return carry
        lax.fori_loop(0, NCHUNK // 2, _pair, 0)
        _wait_scatter(0)
        _wait_scatter(1)

        plsc.subcore_barrier()
        pltpu.sync_copy(agg_sh.at[rs], out.at[c * NP + p, rs])


def _sc_aggregate(src_idx, dst_idx, hm_flat, wp_flat):
    mesh = plsc.VectorSubcoreMesh(core_axis_name="c", subcore_axis_name="s")
    f = pl.kernel(
        _sc_body,
        out_type=jax.ShapeDtypeStruct((NC * NP, NPAD, FC), jnp.float32),
        mesh=mesh,
        scratch_types=[
            pltpu.VMEM((NCHUNK, CHUNK), jnp.int32),
            pltpu.VMEM((NCHUNK, CHUNK), jnp.int32),
            pltpu.VMEM((2, CHUNK, FC), jnp.bfloat16),
            pltpu.VMEM((2, CHUNK, FC), jnp.bfloat16),
            pltpu.VMEM((2, CHUNK, FC), jnp.float32),
            pltpu.VMEM((ZROWS, FC), jnp.float32),
            pltpu.VMEM_SHARED((NPAD, FC), jnp.float32),
            [pltpu.SemaphoreType.DMA, pltpu.SemaphoreType.DMA],
            [pltpu.SemaphoreType.DMA, pltpu.SemaphoreType.DMA],
            [pltpu.SemaphoreType.DMA, pltpu.SemaphoreType.DMA],
        ],
        compiler_params=pltpu.CompilerParams(use_tc_tiling_on_sc=False,
                                             needs_layout_passes=False),
    )
    return f(src_idx, dst_idx, hm_flat, wp_flat)


def kernel(x, edge_vec, t, x_atm, edge_index, batch, emb_x, emb_z, gfp_W,
           t_W1, t_b1, t_W2, t_b2,
           sc_W0, msg_W0, sh_W0, z_W0, rad_W1_0, rad_b1_0, rad_W2_0, rad_b2_0,
           sc_W1, msg_W1, sh_W1, z_W1, rad_W1_1, rad_b1_1, rad_W2_1, rad_b2_1,
           sc_W2, msg_W2, sh_W2, z_W2, rad_W1_2, rad_b1_2, rad_W2_2, rad_b2_2,
           W_out):
    h = jnp.concatenate(
        [jnp.broadcast_to(emb_x, (N, emb_x.shape[1])), x], axis=1)
    batch2 = batch.reshape(N, 1).astype(jnp.int32)
    src = edge_index[0].astype(jnp.int32)
    dst = edge_index[1].astype(jnp.int32)
    offs = (jnp.arange(NC * NP, dtype=jnp.int32) * N).reshape(NC, NP, 1)
    src_idx = (src[None, None, :] + offs).reshape(NC, NP, NS, NCHUNK, CHUNK)
    dst_idx = dst.reshape(NS, NCHUNK, CHUNK)
    S1 = jnp.asarray(np.kron(np.eye(32, dtype=np.float32),
                             np.ones((1, 3), np.float32)))
    S2 = jnp.asarray(np.kron(np.eye(32, dtype=np.float32),
                             np.ones((1, 5), np.float32)))
    W_perm = jnp.transpose(W_out, (2, 0, 1))

    within = np.empty((32,), np.int64)
    within[0::2] = np.arange(16)
    within[1::2] = 16 + np.arange(16)
    PERM = (np.arange(GIN // 32)[:, None] * 32 + within[None, :]).reshape(-1)

    te = _time_embed(t, gfp_W, t_W1, t_b1, t_W2, t_b2)

    layer_w = [
        (sc_W0, msg_W0[:, PERM], sh_W0[:, PERM], z_W0,
         rad_W1_0, rad_b1_0, rad_W2_0[:, PERM], rad_b2_0[PERM]),
        (sc_W1, msg_W1[:, PERM], sh_W1[:, PERM], z_W1,
         rad_W1_1, rad_b1_1, rad_W2_1[:, PERM], rad_b2_1[PERM]),
        (sc_W2, msg_W2[:, PERM], sh_W2[:, PERM], z_W2,
         rad_W1_2, rad_b1_2, rad_W2_2[:, PERM], rad_b2_2[PERM]),
    ]

    hm_flat = _hm0(h, layer_w[0][1]).reshape(NC * NP * N, FC)
    for l in range(3):
        sc_W, msg_W, sh_W, z_W, rW1, rb1, rW2, rb2 = layer_w[l]
        wp_flat = _edge_wprod(edge_vec, rW1, rb1, rW2, rb2,
                              sh_W).reshape(NC * NP * E, FC)
        agg6 = _sc_aggregate(src_idx, dst_idx, hm_flat, wp_flat)
        aggs = [agg6[q, :N] for q in range(NC * NP)]
        if l < 2:
            h, hm_pair = _node_mid(h, batch2, te, emb_z, aggs,
                                   sc_W, z_W, S1, S2, layer_w[l + 1][1])
            hm_flat = hm_pair.reshape(NC * NP * N, FC)
        else:
            out = _node_last(h, batch2, te, emb_z, aggs,
                             sc_W, z_W, S1, S2, W_perm)
    return out


# speedup vs baseline: 1.5014x; 1.1195x over previous
import numpy as np
import jax
import jax.numpy as jnp
from jax import lax
from jax.experimental import pallas as pl
from jax.experimental.pallas import tpu as pltpu
from jax.experimental.pallas import tpu_sc as plsc

N = 10000
E = 160000
B = 128
IN0 = 19
HID = 320
GIN = 384
NUM_BASIS = 32
MAX_RADIUS = 5.0
NUM_NEIGHBORS = 12.0

NC = 2
NS = 16
NP = 3
FC = GIN // (NC * NP)
FCW = FC // 2
ET = E // NS
CHUNK = 125
NCHUNK = ET // CHUNK
NPAD = 10240
ROWS_PER_TILE = NPAD // NS
ZROWS = 128

_EB = 2000
_NB = 2000


def _sigmoid(x):
    return 1.0 / (1.0 + jnp.exp(-x))


def _silu(x):
    return x * _sigmoid(x)


def _sus(u):
    return jnp.where(u > 0.0, jnp.exp(-1.0 / jnp.where(u > 0.0, u, 1.0)), 0.0)


def _pack_words(blk):
    u = lax.bitcast_convert_type(blk, jnp.uint32)
    r16 = (u + jnp.uint32(0x7FFF) + ((u >> 16) & jnp.uint32(1))) >> 16
    word = r16[:, :FCW] | (r16[:, FCW:] << 16)
    return lax.bitcast_convert_type(word, jnp.float32)


def _te_body(t_ref, gfp_ref, w1_ref, b1_ref, w2_ref, b2_ref, out_ref):
    proj = t_ref[...] * gfp_ref[...] * (2.0 * np.pi)
    te = jnp.concatenate([jnp.sin(proj), jnp.cos(proj)], axis=1)
    te = _silu(te @ w1_ref[...] + b1_ref[...]) @ w2_ref[...] + b2_ref[...]
    out_ref[...] = te


def _time_embed(t, gfp_W, t_W1, t_b1, t_W2, t_b2):
    return pl.pallas_call(
        _te_body,
        out_shape=jax.ShapeDtypeStruct((B, 64), jnp.float32),
    )(t.reshape(B, 1), gfp_W.reshape(1, 64), t_W1, t_b1.reshape(1, 128),
      t_W2, t_b2.reshape(1, 64))


def _edge_body(ev_ref, rw1_ref, rb1_ref, rw2_ref, rb2_ref, shw_ref, out_ref):
    ev = ev_ref[...]
    r = jnp.sqrt(jnp.sum(ev * ev, axis=1, keepdims=True))
    step = MAX_RADIUS / (NUM_BASIS + 1)
    centers = (lax.broadcasted_iota(jnp.int32, (1, NUM_BASIS), 1)
               .astype(jnp.float32) + 1.0) * step
    diff = (r - centers) / step
    hb = (1.14136 * float(np.exp(2.0)) * (NUM_BASIS ** 0.5)) * \
        _sus(diff + 1.0) * _sus(1.0 - diff)
    wr = _silu(hb @ rw1_ref[...] + rb1_ref[...]) @ rw2_ref[...] + rb2_ref[...]
    u = ev / jnp.maximum(r, 1e-9)
    ux, uy, uz = u[:, 0:1], u[:, 1:2], u[:, 2:3]
    s3 = 3.0 ** 0.5
    s5 = 5.0 ** 0.5
    s15 = 15.0 ** 0.5
    cols = (jnp.ones_like(ux), s3 * ux, s3 * uy, s3 * uz,
            s15 * ux * uz, s15 * ux * uy,
            s5 * (uy * uy - 0.5 * (ux * ux + uz * uz)),
            s15 * uy * uz, 0.5 * s15 * (uz * uz - ux * ux))
    shw = shw_ref[...]
    acc = cols[0] * shw[0:1, :]
    for q in range(1, 9):
        acc = acc + cols[q] * shw[q:q + 1, :]
    wp = wr * acc
    for q in range(NC * NP):
        out_ref[q] = _pack_words(wp[:, q * FC:(q + 1) * FC])


def _edge_wprod(edge_vec, rW1, rb1, rW2, rb2, sh_W):
    grid = E // _EB
    full = lambda i: (0, 0)
    return pl.pallas_call(
        _edge_body,
        grid=(grid,),
        in_specs=[
            pl.BlockSpec((_EB, 3), lambda i: (i, 0)),
            pl.BlockSpec((NUM_BASIS, 64), full),
            pl.BlockSpec((1, 64), full),
            pl.BlockSpec((64, GIN), full),
            pl.BlockSpec((1, GIN), full),
            pl.BlockSpec((9, GIN), full),
        ],
        out_specs=pl.BlockSpec((NC * NP, _EB, FCW), lambda i: (0, i, 0)),
        out_shape=jax.ShapeDtypeStruct((NC * NP, E, FCW), jnp.float32),
    )(edge_vec, rW1, rb1.reshape(1, 64), rW2, rb2.reshape(1, GIN), sh_W)


def _hm0_body(h_ref, mw_ref, out_ref):
    hm = h_ref[...] @ mw_ref[...]
    for q in range(NC * NP):
        out_ref[q] = _pack_words(hm[:, q * FC:(q + 1) * FC])


def _hm0(h0, msg_W0):
    grid = N // _NB
    return pl.pallas_call(
        _hm0_body,
        grid=(grid,),
        in_specs=[
            pl.BlockSpec((_NB, IN0), lambda i: (i, 0)),
            pl.BlockSpec((IN0, GIN), lambda i: (0, 0)),
        ],
        out_specs=pl.BlockSpec((NC * NP, _NB, FCW), lambda i: (0, i, 0)),
        out_shape=jax.ShapeDtypeStruct((NC * NP, N, FCW), jnp.float32),
    )(h0, msg_W0)


def _node_update(h_ref, b_ref, te_ref, ez_ref, agg_refs,
                 scw_ref, zw_ref, s1_ref, s2_ref):
    lanes = lax.broadcasted_iota(jnp.int32, (1, B), 1)
    onehot = (b_ref[...] == lanes).astype(jnp.float32)
    z = onehot @ te_ref[...] + ez_ref[...]
    agg = jnp.concatenate([a[...] for a in agg_refs], axis=1) * \
        (1.0 / (NUM_NEIGHBORS ** 0.5))
    pre = h_ref[...] @ scw_ref[...] + z @ zw_ref[...] + agg
    scal = _silu(pre[:, :64])
    gates = _sigmoid(pre[:, 64:128])
    gated = pre[:, 128:]
    g1 = gates[:, :32] @ s1_ref[...]
    g2 = gates[:, 32:64] @ s2_ref[...]
    hn = jnp.concatenate(
        [scal, gated[:, :96] * g1, gated[:, 96:] * g2], axis=1)
    return z, hn


def _node_mid_body(h_ref, b_ref, te_ref, ez_ref, a0, a1, a2, a3, a4, a5,
                   scw_ref, zw_ref, s1_ref, s2_ref, mw_ref,
                   hn_ref, hm_ref):
    _, hn = _node_update(h_ref, b_ref, te_ref, ez_ref,
                         (a0, a1, a2, a3, a4, a5),
                         scw_ref, zw_ref, s1_ref, s2_ref)
    hn_ref[...] = hn
    hm = hn @ mw_ref[...]
    for q in range(NC * NP):
        hm_ref[q] = _pack_words(hm[:, q * FC:(q + 1) * FC])


def _node_last_body(h_ref, b_ref, te_ref, ez_ref, a0, a1, a2, a3, a4, a5,
                    scw_ref, zw_ref, s1_ref, s2_ref, wp_ref, out_ref):
    z, hn = _node_update(h_ref, b_ref, te_ref, ez_ref,
                         (a0, a1, a2, a3, a4, a5),
                         scw_ref, zw_ref, s1_ref, s2_ref)
    outs = []
    for k in range(3):
        tk = hn @ wp_ref[k]
        outs.append(jnp.sum(tk * z, axis=1, keepdims=True))
    out_ref[...] = jnp.concatenate(outs, axis=1)


def _node_specs(d):
    full = lambda i: (0, 0)
    return [
        pl.BlockSpec((_NB, d), lambda i: (i, 0)),
        pl.BlockSpec((_NB, 1), lambda i: (i, 0)),
        pl.BlockSpec((B, 64), full),
        pl.BlockSpec((1, 64), full),
    ] + [pl.BlockSpec((_NB, FC), lambda i: (i, 0))] * (NC * NP) + [
        pl.BlockSpec((d, GIN), full),
        pl.BlockSpec((64, GIN), full),
        pl.BlockSpec((32, 96), full),
        pl.BlockSpec((32, 160), full),
    ]


def _node_mid(h, batch2, te, emb_z, aggs, sc_W, z_W, S1, S2, msg_W_next):
    d = h.shape[1]
    grid = N // _NB
    return pl.pallas_call(
        _node_mid_body,
        grid=(grid,),
        in_specs=_node_specs(d) + [pl.BlockSpec((HID, GIN), lambda i: (0, 0))],
        out_specs=[
            pl.BlockSpec((_NB, HID), lambda i: (i, 0)),
            pl.BlockSpec((NC * NP, _NB, FCW), lambda i: (0, i, 0)),
        ],
        out_shape=[
            jax.ShapeDtypeStruct((N, HID), jnp.float32),
            jax.ShapeDtypeStruct((NC * NP, N, FCW), jnp.float32),
        ],
    )(h, batch2, te, emb_z, *aggs, sc_W, z_W, S1, S2, msg_W_next)


def _node_last(h, batch2, te, emb_z, aggs, sc_W, z_W, S1, S2, W_perm):
    d = h.shape[1]
    grid = N // _NB
    return pl.pallas_call(
        _node_last_body,
        grid=(grid,),
        in_specs=_node_specs(d) + [pl.BlockSpec((3, HID, 64), lambda i: (0, 0, 0))],
        out_specs=pl.BlockSpec((_NB, 3), lambda i: (i, 0)),
        out_shape=jax.ShapeDtypeStruct((N, 3), jnp.float32),
    )(h, batch2, te, emb_z, *aggs, sc_W, z_W, S1, S2, W_perm)


def _sc_body(src_hbm, dst_hbm, hm_hbm, wp_hbm, out,
             idxs, idxd, rows, wbuf, prod, zbuf, agg_sh, semg, semw, sems):
    c = lax.axis_index("c")
    s = lax.axis_index("s")
    rs = pl.ds(s * ROWS_PER_TILE, ROWS_PER_TILE)

    pltpu.sync_copy(dst_hbm.at[s], idxd)

    def _zrow(j, carry):
        for k in range(FC // 16):
            zbuf[j, pl.ds(k * 16, 16)] = jnp.zeros((16,), jnp.float32)
        return carry
    lax.fori_loop(0, ZROWS, _zrow, 0)

    for p in range(NP):
        pltpu.sync_copy(src_hbm.at[c, p, s], idxs)
        for i in range(ROWS_PER_TILE // ZROWS):
            pltpu.sync_copy(
                zbuf, agg_sh.at[pl.ds(s * ROWS_PER_TILE + i * ZROWS, ZROWS)])
        plsc.subcore_barrier()

        ebase = (c * NP + p) * E + s * ET

        def _issue(i, b):
            pltpu.async_copy(hm_hbm.at[idxs.at[i]], rows.at[b], semg[b])
            pltpu.async_copy(wp_hbm.at[pl.ds(ebase + i * CHUNK, CHUNK)],
                             wbuf.at[b], semw[b])

        def _wait(b):
            pltpu.make_async_copy(hm_hbm.at[pl.ds(0, CHUNK)], rows.at[b],
                                  semg[b]).wait()
            pltpu.make_async_copy(wp_hbm.at[pl.ds(0, CHUNK)], wbuf.at[b],
                                  semw[b]).wait()

        def _process(i, b):
            _wait(b)

            @plsc.parallel_loop(0, CHUNK, unroll=8)
            def _mul(j):
                for k in range(FC // 32):
                    sl = pl.ds(k * 16, 16)
                    pr = (plsc.bitcast(rows[b, j, sl], jnp.bfloat16) *
                          plsc.bitcast(wbuf[b, j, sl], jnp.bfloat16))
                    lo, hi = plsc.unpack(
                        pr, format=plsc.PackFormat.INTERLEAVED)
                    prod[b, j, pl.ds(k * 32, 16)] = lo
                    prod[b, j, pl.ds(k * 32 + 16, 16)] = hi
            pltpu.async_copy(prod.at[b], agg_sh.at[idxd.at[i]], sems[b],
                             add=True)

        def _wait_scatter(b):
            pltpu.make_async_copy(prod.at[b], agg_sh.at[pl.ds(0, CHUNK)],
                                  sems[b]).wait()

        _issue(0, 0)

        def _pair(g, carry):
            _issue(2 * g + 1, 1)
            _process(2 * g, 0)

            @pl.when(g < NCHUNK // 2 - 1)
            def _():
                _wait_scatter(0)
                _issue(2 * g + 2, 0)
            _process(2 * g + 1, 1)

            @pl.when(g < NCHUNK // 2 - 1)
            def _():
                _wait_scatter(1)
            return carry
        lax.fori_loop(0, NCHUNK // 2, _pair, 0)
        _wait_scatter(0)
        _wait_scatter(1)

        plsc.subcore_barrier()
        pltpu.sync_copy(agg_sh.at[rs], out.at[c * NP + p, rs])


def _sc_aggregate(src_idx, dst_idx, hm_flat, wp_flat):
    mesh = plsc.VectorSubcoreMesh(core_axis_name="c", subcore_axis_name="s")
    f = pl.kernel(
        _sc_body,
        out_type=jax.ShapeDtypeStruct((NC * NP, NPAD, FC), jnp.float32),
        mesh=mesh,
        scratch_types=[
            pltpu.VMEM((NCHUNK, CHUNK), jnp.int32),
            pltpu.VMEM((NCHUNK, CHUNK), jnp.int32),
            pltpu.VMEM((2, CHUNK, FCW), jnp.float32),
            pltpu.VMEM((2, CHUNK, FCW), jnp.float32),
            pltpu.VMEM((2, CHUNK, FC), jnp.float32),
            pltpu.VMEM((ZROWS, FC), jnp.float32),
            pltpu.VMEM_SHARED((NPAD, FC), jnp.float32),
            [pltpu.SemaphoreType.DMA, pltpu.SemaphoreType.DMA],
            [pltpu.SemaphoreType.DMA, pltpu.SemaphoreType.DMA],
            [pltpu.SemaphoreType.DMA, pltpu.SemaphoreType.DMA],
        ],
        compiler_params=pltpu.CompilerParams(use_tc_tiling_on_sc=False,
                                             needs_layout_passes=False),
    )
    return f(src_idx, dst_idx, hm_flat, wp_flat)


def kernel(x, edge_vec, t, x_atm, edge_index, batch, emb_x, emb_z, gfp_W,
           t_W1, t_b1, t_W2, t_b2,
           sc_W0, msg_W0, sh_W0, z_W0, rad_W1_0, rad_b1_0, rad_W2_0, rad_b2_0,
           sc_W1, msg_W1, sh_W1, z_W1, rad_W1_1, rad_b1_1, rad_W2_1, rad_b2_1,
           sc_W2, msg_W2, sh_W2, z_W2, rad_W1_2, rad_b1_2, rad_W2_2, rad_b2_2,
           W_out):
    h = jnp.concatenate(
        [jnp.broadcast_to(emb_x, (N, emb_x.shape[1])), x], axis=1)
    batch2 = batch.reshape(N, 1).astype(jnp.int32)
    src = edge_index[0].astype(jnp.int32)
    dst = edge_index[1].astype(jnp.int32)
    offs = (jnp.arange(NC * NP, dtype=jnp.int32) * N).reshape(NC, NP, 1)
    src_idx = (src[None, None, :] + offs).reshape(NC, NP, NS, NCHUNK, CHUNK)
    dst_idx = dst.reshape(NS, NCHUNK, CHUNK)
    S1 = jnp.asarray(np.kron(np.eye(32, dtype=np.float32),
                             np.ones((1, 3), np.float32)))
    S2 = jnp.asarray(np.kron(np.eye(32, dtype=np.float32),
                             np.ones((1, 5), np.float32)))
    W_perm = jnp.transpose(W_out, (2, 0, 1))

    per64 = np.concatenate([np.arange(16), np.arange(32, 48),
                            np.arange(16, 32), np.arange(48, 64)])
    PERM = (np.arange(GIN // 64)[:, None] * 64 + per64[None, :]).reshape(-1)

    te = _time_embed(t, gfp_W, t_W1, t_b1, t_W2, t_b2)

    layer_w = [
        (sc_W0, msg_W0[:, PERM], sh_W0[:, PERM], z_W0,
         rad_W1_0, rad_b1_0, rad_W2_0[:, PERM], rad_b2_0[PERM]),
        (sc_W1, msg_W1[:, PERM], sh_W1[:, PERM], z_W1,
         rad_W1_1, rad_b1_1, rad_W2_1[:, PERM], rad_b2_1[PERM]),
        (sc_W2, msg_W2[:, PERM], sh_W2[:, PERM], z_W2,
         rad_W1_2, rad_b1_2, rad_W2_2[:, PERM], rad_b2_2[PERM]),
    ]

    hm_flat = _hm0(h, layer_w[0][1]).reshape(NC * NP * N, FCW)
    for l in range(3):
        sc_W, msg_W, sh_W, z_W, rW1, rb1, rW2, rb2 = layer_w[l]
        wp_flat = _edge_wprod(edge_vec, rW1, rb1, rW2, rb2,
                              sh_W).reshape(NC * NP * E, FCW)
        agg6 = _sc_aggregate(src_idx, dst_idx, hm_flat, wp_flat)
        aggs = [agg6[q, :N] for q in range(NC * NP)]
        if l < 2:
            h, hm_pair = _node_mid(h, batch2, te, emb_z, aggs,
                                   sc_W, z_W, S1, S2, layer_w[l + 1][1])
            hm_flat = hm_pair.reshape(NC * NP * N, FCW)
        else:
            out = _node_last(h, batch2, te, emb_z, aggs,
                             sc_W, z_W, S1, S2, W_perm)
    return out


# speedup vs baseline: 1.6850x; 1.1222x over previous
import numpy as np
import jax
import jax.numpy as jnp
from jax import lax
from jax.experimental import pallas as pl
from jax.experimental.pallas import tpu as pltpu
from jax.experimental.pallas import tpu_sc as plsc

N = 10000
E = 160000
B = 128
IN0 = 19
HID = 320
GIN = 384
NUM_BASIS = 32
MAX_RADIUS = 5.0
NUM_NEIGHBORS = 12.0

NC = 2
NS = 16
NP = 3
FC = GIN // (NC * NP)
FCW = FC // 2
ET = E // NS
CHUNK = 125
NCHUNK = ET // CHUNK
NPAD = 10240
ROWS_PER_TILE = NPAD // NS
ZROWS = 128

_EB = 2000
_NB = 2000


def _sigmoid(x):
    return 1.0 / (1.0 + jnp.exp(-x))


def _silu(x):
    return x * _sigmoid(x)


def _sus(u):
    return jnp.where(u > 0.0, jnp.exp(-1.0 / jnp.where(u > 0.0, u, 1.0)), 0.0)


def _pack_words(blk):
    u = lax.bitcast_convert_type(blk, jnp.uint32) + jnp.uint32(0x8000)
    word = (u[:, :FCW] >> 16) | (u[:, FCW:] & jnp.uint32(0xFFFF0000))
    return lax.bitcast_convert_type(word, jnp.float32)


def _te_body(t_ref, gfp_ref, w1_ref, b1_ref, w2_ref, b2_ref, out_ref):
    proj = t_ref[...] * gfp_ref[...] * (2.0 * np.pi)
    te = jnp.concatenate([jnp.sin(proj), jnp.cos(proj)], axis=1)
    te = _silu(te @ w1_ref[...] + b1_ref[...]) @ w2_ref[...] + b2_ref[...]
    out_ref[...] = te


def _time_embed(t, gfp_W, t_W1, t_b1, t_W2, t_b2):
    return pl.pallas_call(
        _te_body,
        out_shape=jax.ShapeDtypeStruct((B, 64), jnp.float32),
    )(t.reshape(B, 1), gfp_W.reshape(1, 64), t_W1, t_b1.reshape(1, 128),
      t_W2, t_b2.reshape(1, 64))


def _edge_body(ev_ref, rw1_ref, rb1_ref, rw2_ref, rb2_ref, shw_ref, out_ref):
    ev = ev_ref[...]
    r = jnp.sqrt(jnp.sum(ev * ev, axis=1, keepdims=True))
    step = MAX_RADIUS / (NUM_BASIS + 1)
    centers = (lax.broadcasted_iota(jnp.int32, (1, NUM_BASIS), 1)
               .astype(jnp.float32) + 1.0) * step
    diff = (r - centers) / step
    hb = (1.14136 * float(np.exp(2.0)) * (NUM_BASIS ** 0.5)) * \
        _sus(diff + 1.0) * _sus(1.0 - diff)
    wr = _silu(hb @ rw1_ref[...] + rb1_ref[...]) @ rw2_ref[...] + rb2_ref[...]
    u = ev / jnp.maximum(r, 1e-9)
    ux, uy, uz = u[:, 0:1], u[:, 1:2], u[:, 2:3]
    s3 = 3.0 ** 0.5
    s5 = 5.0 ** 0.5
    s15 = 15.0 ** 0.5
    sh9 = jnp.concatenate(
        [jnp.ones_like(ux), s3 * ux, s3 * uy, s3 * uz,
         s15 * ux * uz, s15 * ux * uy,
         s5 * (uy * uy - 0.5 * (ux * ux + uz * uz)),
         s15 * uy * uz, 0.5 * s15 * (uz * uz - ux * ux)], axis=1)
    acc = sh9 @ shw_ref[...]
    wp = wr * acc
    for q in range(NC * NP):
        out_ref[q] = _pack_words(wp[:, q * FC:(q + 1) * FC])


def _edge_wprod(edge_vec, rW1, rb1, rW2, rb2, sh_W):
    grid = E // _EB
    full = lambda i: (0, 0)
    return pl.pallas_call(
        _edge_body,
        grid=(grid,),
        in_specs=[
            pl.BlockSpec((_EB, 3), lambda i: (i, 0)),
            pl.BlockSpec((NUM_BASIS, 64), full),
            pl.BlockSpec((1, 64), full),
            pl.BlockSpec((64, GIN), full),
            pl.BlockSpec((1, GIN), full),
            pl.BlockSpec((9, GIN), full),
        ],
        out_specs=pl.BlockSpec((NC * NP, _EB, FCW), lambda i: (0, i, 0)),
        out_shape=jax.ShapeDtypeStruct((NC * NP, E, FCW), jnp.float32),
    )(edge_vec, rW1, rb1.reshape(1, 64), rW2, rb2.reshape(1, GIN), sh_W)


def _hm0_body(h_ref, mw_ref, out_ref):
    hm = h_ref[...] @ mw_ref[...]
    for q in range(NC * NP):
        out_ref[q] = _pack_words(hm[:, q * FC:(q + 1) * FC])


def _hm0(h0, msg_W0):
    grid = N // _NB
    return pl.pallas_call(
        _hm0_body,
        grid=(grid,),
        in_specs=[
            pl.BlockSpec((_NB, IN0), lambda i: (i, 0)),
            pl.BlockSpec((IN0, GIN), lambda i: (0, 0)),
        ],
        out_specs=pl.BlockSpec((NC * NP, _NB, FCW), lambda i: (0, i, 0)),
        out_shape=jax.ShapeDtypeStruct((NC * NP, N, FCW), jnp.float32),
    )(h0, msg_W0)


def _node_update(h_ref, b_ref, te_ref, ez_ref, agg_refs,
                 scw_ref, zw_ref, s1_ref, s2_ref):
    lanes = lax.broadcasted_iota(jnp.int32, (1, B), 1)
    onehot = (b_ref[...] == lanes).astype(jnp.float32)
    z = onehot @ te_ref[...] + ez_ref[...]
    agg = jnp.concatenate([a[...] for a in agg_refs], axis=1) * \
        (1.0 / (NUM_NEIGHBORS ** 0.5))
    pre = h_ref[...] @ scw_ref[...] + z @ zw_ref[...] + agg
    scal = _silu(pre[:, :64])
    gates = _sigmoid(pre[:, 64:128])
    gated = pre[:, 128:]
    g1 = gates[:, :32] @ s1_ref[...]
    g2 = gates[:, 32:64] @ s2_ref[...]
    hn = jnp.concatenate(
        [scal, gated[:, :96] * g1, gated[:, 96:] * g2], axis=1)
    return z, hn


def _node_mid_body(h_ref, b_ref, te_ref, ez_ref, a0, a1, a2, a3, a4, a5,
                   scw_ref, zw_ref, s1_ref, s2_ref, mw_ref,
                   hn_ref, hm_ref):
    _, hn = _node_update(h_ref, b_ref, te_ref, ez_ref,
                         (a0, a1, a2, a3, a4, a5),
                         scw_ref, zw_ref, s1_ref, s2_ref)
    hn_ref[...] = hn
    hm = hn @ mw_ref[...]
    for q in range(NC * NP):
        hm_ref[q] = _pack_words(hm[:, q * FC:(q + 1) * FC])


def _node_last_body(h_ref, b_ref, te_ref, ez_ref, a0, a1, a2, a3, a4, a5,
                    scw_ref, zw_ref, s1_ref, s2_ref, wp_ref, out_ref):
    z, hn = _node_update(h_ref, b_ref, te_ref, ez_ref,
                         (a0, a1, a2, a3, a4, a5),
                         scw_ref, zw_ref, s1_ref, s2_ref)
    outs = []
    for k in range(3):
        tk = hn @ wp_ref[k]
        outs.append(jnp.sum(tk * z, axis=1, keepdims=True))
    out_ref[...] = jnp.concatenate(outs, axis=1)


def _node_specs(d):
    full = lambda i: (0, 0)
    return [
        pl.BlockSpec((_NB, d), lambda i: (i, 0)),
        pl.BlockSpec((_NB, 1), lambda i: (i, 0)),
        pl.BlockSpec((B, 64), full),
        pl.BlockSpec((1, 64), full),
    ] + [pl.BlockSpec((_NB, FC), lambda i: (i, 0))] * (NC * NP) + [
        pl.BlockSpec((d, GIN), full),
        pl.BlockSpec((64, GIN), full),
        pl.BlockSpec((32, 96), full),
        pl.BlockSpec((32, 160), full),
    ]


def _node_mid(h, batch2, te, emb_z, aggs, sc_W, z_W, S1, S2, msg_W_next):
    d = h.shape[1]
    grid = N // _NB
    return pl.pallas_call(
        _node_mid_body,
        grid=(grid,),
        in_specs=_node_specs(d) + [pl.BlockSpec((HID, GIN), lambda i: (0, 0))],
        out_specs=[
            pl.BlockSpec((_NB, HID), lambda i: (i, 0)),
            pl.BlockSpec((NC * NP, _NB, FCW), lambda i: (0, i, 0)),
        ],
        out_shape=[
            jax.ShapeDtypeStruct((N, HID), jnp.float32),
            jax.ShapeDtypeStruct((NC * NP, N, FCW), jnp.float32),
        ],
    )(h, batch2, te, emb_z, *aggs, sc_W, z_W, S1, S2, msg_W_next)


def _node_last(h, batch2, te, emb_z, aggs, sc_W, z_W, S1, S2, W_perm):
    d = h.shape[1]
    grid = N // _NB
    return pl.pallas_call(
        _node_last_body,
        grid=(grid,),
        in_specs=_node_specs(d) + [pl.BlockSpec((3, HID, 64), lambda i: (0, 0, 0))],
        out_specs=pl.BlockSpec((_NB, 3), lambda i: (i, 0)),
        out_shape=jax.ShapeDtypeStruct((N, 3), jnp.float32),
    )(h, batch2, te, emb_z, *aggs, sc_W, z_W, S1, S2, W_perm)


def _sc_body(src_hbm, dst_hbm, hm_hbm, wp_hbm, out,
             idxs, idxd, rows, wbuf, prod, zbuf, agg_sh, semg, semw, sems):
    c = lax.axis_index("c")
    s = lax.axis_index("s")
    rs = pl.ds(s * ROWS_PER_TILE, ROWS_PER_TILE)

    pltpu.sync_copy(dst_hbm.at[s], idxd)

    def _zrow(j, carry):
        for k in range(FC // 16):
            zbuf[j, pl.ds(k * 16, 16)] = jnp.zeros((16,), jnp.float32)
        return carry
    lax.fori_loop(0, ZROWS, _zrow, 0)

    for p in range(NP):
        pltpu.sync_copy(src_hbm.at[c, p, s], idxs)
        for i in range(ROWS_PER_TILE // ZROWS):
            pltpu.sync_copy(
                zbuf, agg_sh.at[pl.ds(s * ROWS_PER_TILE + i * ZROWS, ZROWS)])
        plsc.subcore_barrier()

        ebase = (c * NP + p) * E + s * ET

        def _issue(i, b):
            pltpu.async_copy(hm_hbm.at[idxs.at[i]], rows.at[b], semg[b])
            pltpu.async_copy(wp_hbm.at[pl.ds(ebase + i * CHUNK, CHUNK)],
                             wbuf.at[b], semw[b])

        def _wait(b):
            pltpu.make_async_copy(hm_hbm.at[pl.ds(0, CHUNK)], rows.at[b],
                                  semg[b]).wait()
            pltpu.make_async_copy(wp_hbm.at[pl.ds(0, CHUNK)], wbuf.at[b],
                                  semw[b]).wait()

        def _process(i, b):
            _wait(b)

            @plsc.parallel_loop(0, CHUNK, unroll=8)
            def _mul(j):
                for k in range(FC // 32):
                    sl = pl.ds(k * 16, 16)
                    pr = (plsc.bitcast(rows[b, j, sl], jnp.bfloat16) *
                          plsc.bitcast(wbuf[b, j, sl], jnp.bfloat16))
                    lo, hi = plsc.unpack(
                        pr, format=plsc.PackFormat.INTERLEAVED)
                    prod[b, j, pl.ds(k * 32, 16)] = lo
                    prod[b, j, pl.ds(k * 32 + 16, 16)] = hi
            pltpu.async_copy(prod.at[b], agg_sh.at[idxd.at[i]], sems[b],
                             add=True)

        def _wait_scatter(b):
            pltpu.make_async_copy(prod.at[b], agg_sh.at[pl.ds(0, CHUNK)],
                                  sems[b]).wait()

        _issue(0, 0)

        def _pair(g, carry):
            _issue(2 * g + 1, 1)
            _process(2 * g, 0)

            @pl.when(g < NCHUNK // 2 - 1)
            def _():
                _wait_scatter(0)
                _issue(2 * g + 2, 0)
            _process(2 * g + 1, 1)

            @pl.when(g < NCHUNK // 2 - 1)
            def _():
                _wait_scatter(1)
            return carry
        lax.fori_loop(0, NCHUNK // 2, _pair, 0)
        _wait_scatter(0)
        _wait_scatter(1)

        plsc.subcore_barrier()
        pltpu.sync_copy(agg_sh.at[rs], out.at[c * NP + p, rs])


def _sc_aggregate(src_idx, dst_idx, hm_flat, wp_flat):
    mesh = plsc.VectorSubcoreMesh(core_axis_name="c", subcore_axis_name="s")
    f = pl.kernel(
        _sc_body,
        out_type=jax.ShapeDtypeStruct((NC * NP, NPAD, FC), jnp.float32),
        mesh=mesh,
        scratch_types=[
            pltpu.VMEM((NCHUNK, CHUNK), jnp.int32),
            pltpu.VMEM((NCHUNK, CHUNK), jnp.int32),
            pltpu.VMEM((2, CHUNK, FCW), jnp.float32),
            pltpu.VMEM((2, CHUNK, FCW), jnp.float32),
            pltpu.VMEM((2, CHUNK, FC), jnp.float32),
            pltpu.VMEM((ZROWS, FC), jnp.float32),
            pltpu.VMEM_SHARED((NPAD, FC), jnp.float32),
            [pltpu.SemaphoreType.DMA, pltpu.SemaphoreType.DMA],
            [pltpu.SemaphoreType.DMA, pltpu.SemaphoreType.DMA],
            [pltpu.SemaphoreType.DMA, pltpu.SemaphoreType.DMA],
        ],
        compiler_params=pltpu.CompilerParams(use_tc_tiling_on_sc=False,
                                             needs_layout_passes=False),
    )
    return f(src_idx, dst_idx, hm_flat, wp_flat)


def kernel(x, edge_vec, t, x_atm, edge_index, batch, emb_x, emb_z, gfp_W,
           t_W1, t_b1, t_W2, t_b2,
           sc_W0, msg_W0, sh_W0, z_W0, rad_W1_0, rad_b1_0, rad_W2_0, rad_b2_0,
           sc_W1, msg_W1, sh_W1, z_W1, rad_W1_1, rad_b1_1, rad_W2_1, rad_b2_1,
           sc_W2, msg_W2, sh_W2, z_W2, rad_W1_2, rad_b1_2, rad_W2_2, rad_b2_2,
           W_out):
    h = jnp.concatenate(
        [jnp.broadcast_to(emb_x, (N, emb_x.shape[1])), x], axis=1)
    batch2 = batch.reshape(N, 1).astype(jnp.int32)
    src = edge_index[0].astype(jnp.int32)
    dst = edge_index[1].astype(jnp.int32)
    offs = (jnp.arange(NC * NP, dtype=jnp.int32) * N).reshape(NC, NP, 1)
    src_idx = (src[None, None, :] + offs).reshape(NC, NP, NS, NCHUNK, CHUNK)
    dst_idx = dst.reshape(NS, NCHUNK, CHUNK)
    S1 = jnp.asarray(np.kron(np.eye(32, dtype=np.float32),
                             np.ones((1, 3), np.float32)))
    S2 = jnp.asarray(np.kron(np.eye(32, dtype=np.float32),
                             np.ones((1, 5), np.float32)))
    W_perm = jnp.transpose(W_out, (2, 0, 1))

    per64 = np.concatenate([np.arange(16), np.arange(32, 48),
                            np.arange(16, 32), np.arange(48, 64)])
    PERM = (np.arange(GIN // 64)[:, None] * 64 + per64[None, :]).reshape(-1)

    te = _time_embed(t, gfp_W, t_W1, t_b1, t_W2, t_b2)

    layer_w = [
        (sc_W0, msg_W0[:, PERM], sh_W0[:, PERM], z_W0,
         rad_W1_0, rad_b1_0, rad_W2_0[:, PERM], rad_b2_0[PERM]),
        (sc_W1, msg_W1[:, PERM], sh_W1[:, PERM], z_W1,
         rad_W1_1, rad_b1_1, rad_W2_1[:, PERM], rad_b2_1[PERM]),
        (sc_W2, msg_W2[:, PERM], sh_W2[:, PERM], z_W2,
         rad_W1_2, rad_b1_2, rad_W2_2[:, PERM], rad_b2_2[PERM]),
    ]

    hm_flat = _hm0(h, layer_w[0][1]).reshape(NC * NP * N, FCW)
    for l in range(3):
        sc_W, msg_W, sh_W, z_W, rW1, rb1, rW2, rb2 = layer_w[l]
        wp_flat = _edge_wprod(edge_vec, rW1, rb1, rW2, rb2,
                              sh_W).reshape(NC * NP * E, FCW)
        agg6 = _sc_aggregate(src_idx, dst_idx, hm_flat, wp_flat)
        aggs = [agg6[q, :N] for q in range(NC * NP)]
        if l < 2:
            h, hm_pair = _node_mid(h, batch2, te, emb_z, aggs,
                                   sc_W, z_W, S1, S2, layer_w[l + 1][1])
            hm_flat = hm_pair.reshape(NC * NP * N, FCW)
        else:
            out = _node_last(h, batch2, te, emb_z, aggs,
                             sc_W, z_W, S1, S2, W_perm)
    return out


# speedup vs baseline: 1.6867x; 1.0010x over previous
import numpy as np
import jax
import jax.numpy as jnp
from jax import lax
from jax.experimental import pallas as pl
from jax.experimental.pallas import tpu as pltpu
from jax.experimental.pallas import tpu_sc as plsc

N = 10000
E = 160000
B = 128
IN0 = 19
HID = 320
GIN = 384
NUM_BASIS = 32
MAX_RADIUS = 5.0
NUM_NEIGHBORS = 12.0

NC = 2
NS = 16
NP = 3
FC = GIN // (NC * NP)
FCW = FC // 2
ET = E // NS
CHUNK = 125
NCHUNK = ET // CHUNK
NPAD = 10240
ROWS_PER_TILE = NPAD // NS
ZROWS = 128

_EB = 2000
_NB = 2000


def _sigmoid(x):
    return 1.0 / (1.0 + jnp.exp(-x))


def _silu(x):
    return x * _sigmoid(x)


def _sus(u):
    return jnp.where(u > 0.0, jnp.exp(-1.0 / jnp.where(u > 0.0, u, 1.0)), 0.0)


def _pack_words(blk):
    u = lax.bitcast_convert_type(blk, jnp.uint32) + jnp.uint32(0x8000)
    word = (u[:, :FCW] >> 16) | (u[:, FCW:] & jnp.uint32(0xFFFF0000))
    return lax.bitcast_convert_type(word, jnp.float32)


def _te_body(t_ref, gfp_ref, w1_ref, b1_ref, w2_ref, b2_ref, out_ref):
    proj = t_ref[...] * gfp_ref[...] * (2.0 * np.pi)
    te = jnp.concatenate([jnp.sin(proj), jnp.cos(proj)], axis=1)
    te = _silu(te @ w1_ref[...] + b1_ref[...]) @ w2_ref[...] + b2_ref[...]
    out_ref[...] = te


def _time_embed(t, gfp_W, t_W1, t_b1, t_W2, t_b2):
    return pl.pallas_call(
        _te_body,
        out_shape=jax.ShapeDtypeStruct((B, 64), jnp.float32),
    )(t.reshape(B, 1), gfp_W.reshape(1, 64), t_W1, t_b1.reshape(1, 128),
      t_W2, t_b2.reshape(1, 64))


def _edge_body(ev_ref, rw1_ref, rb1_ref, rw2_ref, rb2_ref, shw_ref, out_ref):
    ev = ev_ref[...]
    r = jnp.sqrt(jnp.sum(ev * ev, axis=1, keepdims=True))
    step = MAX_RADIUS / (NUM_BASIS + 1)
    centers = (lax.broadcasted_iota(jnp.int32, (1, NUM_BASIS), 1)
               .astype(jnp.float32) + 1.0) * step
    diff = (r - centers) / step
    hb = (1.14136 * float(np.exp(2.0)) * (NUM_BASIS ** 0.5)) * \
        _sus(diff + 1.0) * _sus(1.0 - diff)
    wr = _silu(hb @ rw1_ref[...] + rb1_ref[...]) @ rw2_ref[...] + rb2_ref[...]
    u = ev / jnp.maximum(r, 1e-9)
    ux, uy, uz = u[:, 0:1], u[:, 1:2], u[:, 2:3]
    s3 = 3.0 ** 0.5
    s5 = 5.0 ** 0.5
    s15 = 15.0 ** 0.5
    sh9 = jnp.concatenate(
        [jnp.ones_like(ux), s3 * ux, s3 * uy, s3 * uz,
         s15 * ux * uz, s15 * ux * uy,
         s5 * (uy * uy - 0.5 * (ux * ux + uz * uz)),
         s15 * uy * uz, 0.5 * s15 * (uz * uz - ux * ux)], axis=1)
    acc = sh9 @ shw_ref[...]
    wp = wr * acc
    for q in range(NC * NP):
        out_ref[q] = _pack_words(wp[:, q * FC:(q + 1) * FC])


def _edge_wprod(edge_vec, rW1, rb1, rW2, rb2, sh_W):
    grid = E // _EB
    full = lambda i: (0, 0)
    return pl.pallas_call(
        _edge_body,
        grid=(grid,),
        in_specs=[
            pl.BlockSpec((_EB, 3), lambda i: (i, 0)),
            pl.BlockSpec((NUM_BASIS, 64), full),
            pl.BlockSpec((1, 64), full),
            pl.BlockSpec((64, GIN), full),
            pl.BlockSpec((1, GIN), full),
            pl.BlockSpec((9, GIN), full),
        ],
        out_specs=pl.BlockSpec((NC * NP, _EB, FCW), lambda i: (0, i, 0)),
        out_shape=jax.ShapeDtypeStruct((NC * NP, E, FCW), jnp.float32),
    )(edge_vec, rW1, rb1.reshape(1, 64), rW2, rb2.reshape(1, GIN), sh_W)


def _hm0_body(h_ref, mw_ref, out_ref):
    hm = h_ref[...] @ mw_ref[...]
    for q in range(NC * NP):
        out_ref[q] = _pack_words(hm[:, q * FC:(q + 1) * FC])


def _hm0(h0, msg_W0):
    grid = N // _NB
    return pl.pallas_call(
        _hm0_body,
        grid=(grid,),
        in_specs=[
            pl.BlockSpec((_NB, IN0), lambda i: (i, 0)),
            pl.BlockSpec((IN0, GIN), lambda i: (0, 0)),
        ],
        out_specs=pl.BlockSpec((NC * NP, _NB, FCW), lambda i: (0, i, 0)),
        out_shape=jax.ShapeDtypeStruct((NC * NP, N, FCW), jnp.float32),
    )(h0, msg_W0)


def _node_update(h_ref, b_ref, te_ref, ez_ref, agg_refs,
                 scw_ref, zw_ref, s1_ref, s2_ref):
    lanes = lax.broadcasted_iota(jnp.int32, (1, B), 1)
    onehot = (b_ref[...] == lanes).astype(jnp.float32)
    z = onehot @ te_ref[...] + ez_ref[...]
    agg = jnp.concatenate([a[...] for a in agg_refs], axis=1) * \
        (1.0 / (NUM_NEIGHBORS ** 0.5))
    pre = h_ref[...] @ scw_ref[...] + z @ zw_ref[...] + agg
    scal = _silu(pre[:, :64])
    gates = _sigmoid(pre[:, 64:128])
    gated = pre[:, 128:]
    g1 = gates[:, :32] @ s1_ref[...]
    g2 = gates[:, 32:64] @ s2_ref[...]
    hn = jnp.concatenate(
        [scal, gated[:, :96] * g1, gated[:, 96:] * g2], axis=1)
    return z, hn


def _node_mid_body(h_ref, b_ref, te_ref, ez_ref, a0, a1, a2, a3, a4, a5,
                   scw_ref, zw_ref, s1_ref, s2_ref, mw_ref,
                   hn_ref, hm_ref):
    _, hn = _node_update(h_ref, b_ref, te_ref, ez_ref,
                         (a0, a1, a2, a3, a4, a5),
                         scw_ref, zw_ref, s1_ref, s2_ref)
    hn_ref[...] = hn
    hm = hn @ mw_ref[...]
    for q in range(NC * NP):
        hm_ref[q] = _pack_words(hm[:, q * FC:(q + 1) * FC])


def _node_last_body(h_ref, b_ref, te_ref, ez_ref, a0, a1, a2, a3, a4, a5,
                    scw_ref, zw_ref, s1_ref, s2_ref, wp_ref, out_ref):
    z, hn = _node_update(h_ref, b_ref, te_ref, ez_ref,
                         (a0, a1, a2, a3, a4, a5),
                         scw_ref, zw_ref, s1_ref, s2_ref)
    outs = []
    for k in range(3):
        tk = hn @ wp_ref[k]
        outs.append(jnp.sum(tk * z, axis=1, keepdims=True))
    out_ref[...] = jnp.concatenate(outs, axis=1)


def _node_specs(d):
    full = lambda i: (0, 0)
    return [
        pl.BlockSpec((_NB, d), lambda i: (i, 0)),
        pl.BlockSpec((_NB, 1), lambda i: (i, 0)),
        pl.BlockSpec((B, 64), full),
        pl.BlockSpec((1, 64), full),
    ] + [pl.BlockSpec((_NB, FC), lambda i: (i, 0))] * (NC * NP) + [
        pl.BlockSpec((d, GIN), full),
        pl.BlockSpec((64, GIN), full),
        pl.BlockSpec((32, 96), full),
        pl.BlockSpec((32, 160), full),
    ]


def _node_mid(h, batch2, te, emb_z, aggs, sc_W, z_W, S1, S2, msg_W_next):
    d = h.shape[1]
    grid = N // _NB
    return pl.pallas_call(
        _node_mid_body,
        grid=(grid,),
        in_specs=_node_specs(d) + [pl.BlockSpec((HID, GIN), lambda i: (0, 0))],
        out_specs=[
            pl.BlockSpec((_NB, HID), lambda i: (i, 0)),
            pl.BlockSpec((NC * NP, _NB, FCW), lambda i: (0, i, 0)),
        ],
        out_shape=[
            jax.ShapeDtypeStruct((N, HID), jnp.float32),
            jax.ShapeDtypeStruct((NC * NP, N, FCW), jnp.float32),
        ],
    )(h, batch2, te, emb_z, *aggs, sc_W, z_W, S1, S2, msg_W_next)


def _node_last(h, batch2, te, emb_z, aggs, sc_W, z_W, S1, S2, W_perm):
    d = h.shape[1]
    grid = N // _NB
    return pl.pallas_call(
        _node_last_body,
        grid=(grid,),
        in_specs=_node_specs(d) + [pl.BlockSpec((3, HID, 64), lambda i: (0, 0, 0))],
        out_specs=pl.BlockSpec((_NB, 3), lambda i: (i, 0)),
        out_shape=jax.ShapeDtypeStruct((N, 3), jnp.float32),
    )(h, batch2, te, emb_z, *aggs, sc_W, z_W, S1, S2, W_perm)


def _sc_body(src_hbm, dst_hbm, hm_hbm, wp_hbm, out,
             idxs, idxd, rows, wbuf, prod, zbuf, agg_sh, semg, semw, sems):
    c = lax.axis_index("c")
    s = lax.axis_index("s")
    rs = pl.ds(s * ROWS_PER_TILE, ROWS_PER_TILE)

    pltpu.sync_copy(dst_hbm.at[s], idxd)
    pltpu.sync_copy(src_hbm.at[s], idxs)

    def _zrow(j, carry):
        for k in range(FC // 16):
            zbuf[j, pl.ds(k * 16, 16)] = jnp.zeros((16,), jnp.float32)
        return carry
    lax.fori_loop(0, ZROWS, _zrow, 0)

    for p in range(NP):
        qi = c * NP + p
        for i in range(ROWS_PER_TILE // ZROWS):
            pltpu.sync_copy(
                zbuf, agg_sh.at[pl.ds(s * ROWS_PER_TILE + i * ZROWS, ZROWS)])
        plsc.subcore_barrier()

        ebase = s * ET

        def _issue(i, b):
            pltpu.async_copy(hm_hbm.at[qi].at[idxs.at[i]], rows.at[b],
                             semg[b])
            pltpu.async_copy(wp_hbm.at[qi, pl.ds(ebase + i * CHUNK, CHUNK)],
                             wbuf.at[b], semw[b])

        def _wait(b):
            pltpu.make_async_copy(hm_hbm.at[0, pl.ds(0, CHUNK)], rows.at[b],
                                  semg[b]).wait()
            pltpu.make_async_copy(wp_hbm.at[0, pl.ds(0, CHUNK)], wbuf.at[b],
                                  semw[b]).wait()

        def _process(i, b):
            _wait(b)

            @plsc.parallel_loop(0, CHUNK, unroll=8)
            def _mul(j):
                for k in range(FC // 32):
                    sl = pl.ds(k * 16, 16)
                    pr = (plsc.bitcast(rows[b, j, sl], jnp.bfloat16) *
                          plsc.bitcast(wbuf[b, j, sl], jnp.bfloat16))
                    lo, hi = plsc.unpack(
                        pr, format=plsc.PackFormat.INTERLEAVED)
                    prod[b, j, pl.ds(k * 32, 16)] = lo
                    prod[b, j, pl.ds(k * 32 + 16, 16)] = hi
            pltpu.async_copy(prod.at[b], agg_sh.at[idxd.at[i]], sems[b],
                             add=True)

        def _wait_scatter(b):
            pltpu.make_async_copy(prod.at[b], agg_sh.at[pl.ds(0, CHUNK)],
                                  sems[b]).wait()

        _issue(0, 0)

        def _pair(g, carry):
            _issue(2 * g + 1, 1)
            _process(2 * g, 0)

            @pl.when(g < NCHUNK // 2 - 1)
            def _():
                _wait_scatter(0)
                _issue(2 * g + 2, 0)
            _process(2 * g + 1, 1)

            @pl.when(g < NCHUNK // 2 - 1)
            def _():
                _wait_scatter(1)
            return carry
        lax.fori_loop(0, NCHUNK // 2, _pair, 0)
        _wait_scatter(0)
        _wait_scatter(1)

        plsc.subcore_barrier()
        pltpu.sync_copy(agg_sh.at[rs], out.at[c * NP + p, rs])


def _sc_aggregate(src_idx, dst_idx, hm_flat, wp_flat):
    mesh = plsc.VectorSubcoreMesh(core_axis_name="c", subcore_axis_name="s")
    f = pl.kernel(
        _sc_body,
        out_type=jax.ShapeDtypeStruct((NC * NP, NPAD, FC), jnp.float32),
        mesh=mesh,
        scratch_types=[
            pltpu.VMEM((NCHUNK, CHUNK), jnp.int32),
            pltpu.VMEM((NCHUNK, CHUNK), jnp.int32),
            pltpu.VMEM((2, CHUNK, FCW), jnp.float32),
            pltpu.VMEM((2, CHUNK, FCW), jnp.float32),
            pltpu.VMEM((2, CHUNK, FC), jnp.float32),
            pltpu.VMEM((ZROWS, FC), jnp.float32),
            pltpu.VMEM_SHARED((NPAD, FC), jnp.float32),
            [pltpu.SemaphoreType.DMA, pltpu.SemaphoreType.DMA],
            [pltpu.SemaphoreType.DMA, pltpu.SemaphoreType.DMA],
            [pltpu.SemaphoreType.DMA, pltpu.SemaphoreType.DMA],
        ],
        compiler_params=pltpu.CompilerParams(use_tc_tiling_on_sc=False,
                                             needs_layout_passes=False),
    )
    return f(src_idx, dst_idx, hm_flat, wp_flat)


def kernel(x, edge_vec, t, x_atm, edge_index, batch, emb_x, emb_z, gfp_W,
           t_W1, t_b1, t_W2, t_b2,
           sc_W0, msg_W0, sh_W0, z_W0, rad_W1_0, rad_b1_0, rad_W2_0, rad_b2_0,
           sc_W1, msg_W1, sh_W1, z_W1, rad_W1_1, rad_b1_1, rad_W2_1, rad_b2_1,
           sc_W2, msg_W2, sh_W2, z_W2, rad_W1_2, rad_b1_2, rad_W2_2, rad_b2_2,
           W_out):
    h = jnp.concatenate(
        [jnp.broadcast_to(emb_x, (N, emb_x.shape[1])), x], axis=1)
    batch2 = batch.reshape(N, 1).astype(jnp.int32)
    src = edge_index[0].astype(jnp.int32)
    dst = edge_index[1].astype(jnp.int32)
    src_idx = src.reshape(NS, NCHUNK, CHUNK)
    dst_idx = dst.reshape(NS, NCHUNK, CHUNK)
    S1 = jnp.asarray(np.kron(np.eye(32, dtype=np.float32),
                             np.ones((1, 3), np.float32)))
    S2 = jnp.asarray(np.kron(np.eye(32, dtype=np.float32),
                             np.ones((1, 5), np.float32)))
    W_perm = jnp.transpose(W_out, (2, 0, 1))

    per64 = np.concatenate([np.arange(16), np.arange(32, 48),
                            np.arange(16, 32), np.arange(48, 64)])
    PERM = (np.arange(GIN // 64)[:, None] * 64 + per64[None, :]).reshape(-1)

    te = _time_embed(t, gfp_W, t_W1, t_b1, t_W2, t_b2)

    layer_w = [
        (sc_W0, msg_W0[:, PERM], sh_W0[:, PERM], z_W0,
         rad_W1_0, rad_b1_0, rad_W2_0[:, PERM], rad_b2_0[PERM]),
        (sc_W1, msg_W1[:, PERM], sh_W1[:, PERM], z_W1,
         rad_W1_1, rad_b1_1, rad_W2_1[:, PERM], rad_b2_1[PERM]),
        (sc_W2, msg_W2[:, PERM], sh_W2[:, PERM], z_W2,
         rad_W1_2, rad_b1_2, rad_W2_2[:, PERM], rad_b2_2[PERM]),
    ]

    hm3 = _hm0(h, layer_w[0][1])
    for l in range(3):
        sc_W, msg_W, sh_W, z_W, rW1, rb1, rW2, rb2 = layer_w[l]
        wp3 = _edge_wprod(edge_vec, rW1, rb1, rW2, rb2, sh_W)
        agg6 = _sc_aggregate(src_idx, dst_idx, hm3, wp3)
        aggs = [agg6[q, :N] for q in range(NC * NP)]
        if l < 2:
            h, hm3 = _node_mid(h, batch2, te, emb_z, aggs,
                               sc_W, z_W, S1, S2, layer_w[l + 1][1])
        else:
            out = _node_last(h, batch2, te, emb_z, aggs,
                             sc_W, z_W, S1, S2, W_perm)
    return out
